# baseline (speedup 1.0000x reference)
import jax
import jax.numpy as jnp
from jax.experimental import pallas as pl

_B_TILE = 512
_ENSEMBLE = 8


def _fused_kernel(x_ref, w_ref, hbd_ref, ones_ref, out_ref):
    x = x_ref[...]
    det = out_ref.shape[1]
    d_all = jax.nn.sigmoid(
        jnp.dot(x, w_ref[...], preferred_element_type=jnp.float32))
    hm = jnp.dot(d_all, hbd_ref[...], preferred_element_type=jnp.float32)
    m2 = hm - 2.0 * jnp.floor(hm * 0.5)
    crc_all = jnp.dot(m2, ones_ref[...], preferred_element_type=jnp.float32)
    best = d_all[:, :det]
    best_crc = crc_all[:, 0:1]
    for i in range(1, _ENSEMBLE):
        crc = crc_all[:, i:i + 1]
        take = crc < best_crc
        best = jnp.where(take, d_all[:, i * det:(i + 1) * det], best)
        best_crc = jnp.where(take, crc, best_crc)
    out_ref[...] = best


def kernel(x, W, code_h_outer):
    batch, in_len = x.shape
    ens, _, det = W.shape
    h_rows = code_h_outer.shape[0]
    w_flat = W.transpose(1, 0, 2).reshape(in_len, ens * det)
    eye = jnp.eye(ens, dtype=jnp.float32)
    h_bd = jnp.kron(eye, code_h_outer.T)
    ones_bd = jnp.kron(eye, jnp.ones((h_rows, 1), jnp.float32))
    return pl.pallas_call(
        _fused_kernel,
        grid=(batch // _B_TILE,),
        in_specs=[
            pl.BlockSpec((_B_TILE, in_len), lambda i: (i, 0)),
            pl.BlockSpec((in_len, ens * det), lambda i: (0, 0)),
            pl.BlockSpec((ens * det, ens * h_rows), lambda i: (0, 0)),
            pl.BlockSpec((ens * h_rows, ens), lambda i: (0, 0)),
        ],
        out_specs=pl.BlockSpec((_B_TILE, det), lambda i: (i, 0)),
        out_shape=jax.ShapeDtypeStruct((batch, det), jnp.float32),
    )(x, w_flat, h_bd, ones_bd)


# speedup vs baseline: 1.8775x; 1.8775x over previous
import jax
import jax.numpy as jnp
from jax.experimental import pallas as pl

_B_TILE = 1024
_ENSEMBLE = 8


def _fused_kernel(x_ref, w_ref, h_ref, out_ref):
    x = x_ref[...]
    h = h_ref[...]
    det = h.shape[1]
    d_all = jax.nn.sigmoid(
        jnp.dot(x, w_ref[...], preferred_element_type=jnp.float32))
    best = None
    best_crc = None
    for i in range(_ENSEMBLE):
        d = d_all[:, i * det:(i + 1) * det]
        hm = jax.lax.dot_general(
            d, h, (((1,), (1,)), ((), ())),
            preferred_element_type=jnp.float32)
        m2 = hm - 2.0 * jnp.floor(hm * 0.5)
        crc = jnp.sum(m2, axis=1, keepdims=True)
        if i == 0:
            best, best_crc = d, crc
        else:
            take = crc < best_crc
            best = jnp.where(take, d, best)
            best_crc = jnp.where(take, crc, best_crc)
    out_ref[...] = best


def kernel(x, W, code_h_outer):
    batch, in_len = x.shape
    ens, _, det = W.shape
    h_rows = code_h_outer.shape[0]
    w_flat = W.transpose(1, 0, 2).reshape(in_len, ens * det)
    return pl.pallas_call(
        _fused_kernel,
        grid=(batch // _B_TILE,),
        in_specs=[
            pl.BlockSpec((_B_TILE, in_len), lambda i: (i, 0)),
            pl.BlockSpec((in_len, ens * det), lambda i: (0, 0)),
            pl.BlockSpec((h_rows, det), lambda i: (0, 0)),
        ],
        out_specs=pl.BlockSpec((_B_TILE, det), lambda i: (i, 0)),
        out_shape=jax.ShapeDtypeStruct((batch, det), jnp.float32),
    )(x, w_flat, code_h_outer)


# speedup vs baseline: 1.9582x; 1.0430x over previous
import jax
import jax.numpy as jnp
from jax.experimental import pallas as pl

_B_TILE = 2048
_ENSEMBLE = 8


def _fused_kernel(x_ref, w_ref, h_ref, out_ref):
    x = x_ref[...]
    h = h_ref[...]
    det = h.shape[1]
    d_all = jax.nn.sigmoid(
        jnp.dot(x, w_ref[...], preferred_element_type=jnp.float32))
    best = None
    best_crc = None
    for i in range(_ENSEMBLE):
        d = d_all[:, i * det:(i + 1) * det]
        hm = jax.lax.dot_general(
            d, h, (((1,), (1,)), ((), ())),
            preferred_element_type=jnp.float32)
        m2 = hm - 2.0 * jnp.floor(hm * 0.5)
        crc = jnp.sum(m2, axis=1, keepdims=True)
        if i == 0:
            best, best_crc = d, crc
        else:
            take = crc < best_crc
            best = jnp.where(take, d, best)
            best_crc = jnp.where(take, crc, best_crc)
    out_ref[...] = best


def kernel(x, W, code_h_outer):
    batch, in_len = x.shape
    ens, _, det = W.shape
    h_rows = code_h_outer.shape[0]
    w_flat = W.transpose(1, 0, 2).reshape(in_len, ens * det)
    return pl.pallas_call(
        _fused_kernel,
        grid=(batch // _B_TILE,),
        in_specs=[
            pl.BlockSpec((_B_TILE, in_len), lambda i: (i, 0)),
            pl.BlockSpec((in_len, ens * det), lambda i: (0, 0)),
            pl.BlockSpec((h_rows, det), lambda i: (0, 0)),
        ],
        out_specs=pl.BlockSpec((_B_TILE, det), lambda i: (i, 0)),
        out_shape=jax.ShapeDtypeStruct((batch, det), jnp.float32),
    )(x, w_flat, code_h_outer)


# speedup vs baseline: 1.9704x; 1.0063x over previous
import jax
import jax.numpy as jnp
from jax.experimental import pallas as pl

_B_TILE = 4096
_ENSEMBLE = 8


def _fused_kernel(x_ref, w_ref, h_ref, out_ref):
    x = x_ref[...]
    h = h_ref[...]
    det = h.shape[1]
    d_all = jax.nn.sigmoid(
        jnp.dot(x, w_ref[...], preferred_element_type=jnp.float32))
    best = None
    best_crc = None
    for i in range(_ENSEMBLE):
        d = d_all[:, i * det:(i + 1) * det]
        hm = jax.lax.dot_general(
            d, h, (((1,), (1,)), ((), ())),
            preferred_element_type=jnp.float32)
        m2 = hm - 2.0 * jnp.floor(hm * 0.5)
        crc = jnp.sum(m2, axis=1, keepdims=True)
        if i == 0:
            best, best_crc = d, crc
        else:
            take = crc < best_crc
            best = jnp.where(take, d, best)
            best_crc = jnp.where(take, crc, best_crc)
    out_ref[...] = best


def kernel(x, W, code_h_outer):
    batch, in_len = x.shape
    ens, _, det = W.shape
    h_rows = code_h_outer.shape[0]
    w_flat = W.transpose(1, 0, 2).reshape(in_len, ens * det)
    return pl.pallas_call(
        _fused_kernel,
        grid=(batch // _B_TILE,),
        in_specs=[
            pl.BlockSpec((_B_TILE, in_len), lambda i: (i, 0)),
            pl.BlockSpec((in_len, ens * det), lambda i: (0, 0)),
            pl.BlockSpec((h_rows, det), lambda i: (0, 0)),
        ],
        out_specs=pl.BlockSpec((_B_TILE, det), lambda i: (i, 0)),
        out_shape=jax.ShapeDtypeStruct((batch, det), jnp.float32),
    )(x, w_flat, code_h_outer)
